# baseline (speedup 1.0000x reference)
import functools

import jax
import jax.numpy as jnp
from jax.experimental import pallas as pl
from jax.experimental.pallas import tpu as pltpu


def _tc_body(s_ref, l_ref, out_ref):
    i = pl.program_id(0)
    n = pl.num_programs(0)
    s = jnp.exp(s_ref[...])
    pos_mask = l_ref[...] > 0
    denom = jnp.sum(s, axis=1)
    possum = jnp.sum(jnp.where(pos_mask, s, 0.0), axis=1)
    negmax = jnp.max(jnp.where(pos_mask, -jnp.inf, s), axis=1)
    loss = -jnp.log((possum + negmax) / denom + 0.05)
    part = jnp.sum(loss)

    @pl.when(i == 0)
    def _():
        out_ref[0, 0] = part

    @pl.when(i > 0)
    def _():
        out_ref[0, 0] = out_ref[0, 0] + part


def kernel(scores, margin, label):
    del margin
    B, N = scores.shape
    BR = 512
    grid = B // BR
    total = pl.pallas_call(
        _tc_body,
        grid=(grid,),
        in_specs=[
            pl.BlockSpec((BR, N), lambda i: (i, 0)),
            pl.BlockSpec((BR, N), lambda i: (i, 0)),
        ],
        out_specs=pl.BlockSpec(memory_space=pltpu.SMEM),
        out_shape=jax.ShapeDtypeStruct((1, 1), jnp.float32),
    )(scores, label)
    return total[0, 0] / B
